# baseline (speedup 1.0000x reference)
import functools

import jax
import jax.numpy as jnp
from jax import lax
from jax.experimental import pallas as pl
from jax.experimental.pallas import tpu as pltpu
from jax.experimental.pallas import tpu_sc as plsc

NTOKEN = 1000000
EMB_DIM = 64

_info = plsc.get_sparse_core_info()
_NC, _NS = _info.num_cores, _info.num_subcores
_NW = _NC * _NS

_B = 16384 * 20
_BPW = _B // _NW
_C = 512
_NCHUNK = _BPW // _C
_NBUF = 2
_NG = _NCHUNK // _NBUF


def _make_kernel():
    mesh = plsc.VectorSubcoreMesh(core_axis_name="c", subcore_axis_name="s")

    @functools.partial(
        pl.kernel,
        mesh=mesh,
        out_type=jax.ShapeDtypeStruct((_B, EMB_DIM), jnp.float32),
        scratch_types=[
            pltpu.VMEM((_NBUF, _C), jnp.int32),
            pltpu.VMEM((_NBUF, _C, EMB_DIM), jnp.float32),
            pltpu.SemaphoreType.DMA((_NBUF,)),
        ],
        compiler_params=pltpu.CompilerParams(use_tc_tiling_on_sc=False),
    )
    def emb_kernel(table_hbm, idx_hbm, out_hbm, idx_v, rows_v, gsem):
        wid = lax.axis_index("s") * _NC + lax.axis_index("c")
        base = wid * _BPW

        def fire(t, b):
            pltpu.sync_copy(idx_hbm.at[wid, t], idx_v.at[b])

            def row16(q, _):
                r = q * 16
                iv = idx_v[b, pl.ds(r, 16)]
                for u in range(16):
                    pltpu.async_copy(table_hbm.at[pl.ds(iv[u], 1)],
                                     rows_v.at[b, pl.ds(r + u, 1)],
                                     gsem.at[b])
                return _

            lax.fori_loop(0, _C // 16, row16, None)

        for b in range(_NBUF):
            fire(b, b)

        def group(g, _):
            for b in range(_NBUF):
                t = g * _NBUF + b
                pltpu.make_async_copy(table_hbm.at[pl.ds(0, _C)],
                                      rows_v.at[b], gsem.at[b]).wait()
                pltpu.sync_copy(rows_v.at[b],
                                out_hbm.at[pl.ds(base + t * _C, _C)])

                @pl.when(g < _NG - 1)
                def _refill():
                    fire(t + _NBUF, b)
            return _

        lax.fori_loop(0, _NG, group, None)

    return emb_kernel


_emb_kernel = _make_kernel()


@jax.jit
def kernel(x, table):
    idx = x.astype(jnp.int32).reshape(_NW, _NCHUNK, _C)
    out = _emb_kernel(table, idx)
    return out.reshape(x.shape[0], x.shape[1], EMB_DIM)


# speedup vs baseline: 1.2033x; 1.2033x over previous
import functools

import jax
import jax.numpy as jnp
from jax import lax
from jax.experimental import pallas as pl
from jax.experimental.pallas import tpu as pltpu
from jax.experimental.pallas import tpu_sc as plsc

NTOKEN = 1000000
EMB_DIM = 64

_info = plsc.get_sparse_core_info()
_NC, _NS = _info.num_cores, _info.num_subcores
_NW = _NC * _NS

_B = 16384 * 20
_BPW = _B // _NW
_C = 256
_NCHUNK = _BPW // _C
_NBUF = 2
_NG = _NCHUNK // _NBUF


def _make_kernel():
    mesh = plsc.VectorSubcoreMesh(core_axis_name="c", subcore_axis_name="s")

    @functools.partial(
        pl.kernel,
        mesh=mesh,
        out_type=jax.ShapeDtypeStruct((_B, EMB_DIM), jnp.float32),
        scratch_types=[
            pltpu.VMEM((_NBUF, _C), jnp.int32),
            pltpu.VMEM((_NBUF, _C, EMB_DIM), jnp.float32),
            pltpu.SemaphoreType.DMA((_NBUF,)),
        ],
        compiler_params=pltpu.CompilerParams(use_tc_tiling_on_sc=True),
    )
    def emb_kernel(table_hbm, idx_hbm, out_hbm, idx_v, rows_v, gsem):
        wid = lax.axis_index("s") * _NC + lax.axis_index("c")
        base = wid * _BPW

        def fire(t, b):
            pltpu.sync_copy(idx_hbm.at[wid, t], idx_v.at[b])

            def row16(q, _):
                r = q * 16
                iv = idx_v[b, pl.ds(r, 16)]
                for u in range(16):
                    pltpu.async_copy(table_hbm.at[pl.ds(iv[u], 1)],
                                     rows_v.at[b, pl.ds(r + u, 1)],
                                     gsem.at[b])
                return _

            lax.fori_loop(0, _C // 16, row16, None)

        for b in range(_NBUF):
            fire(b, b)

        def group(g, _):
            for b in range(_NBUF):
                t = g * _NBUF + b
                pltpu.make_async_copy(table_hbm.at[pl.ds(0, _C)],
                                      rows_v.at[b], gsem.at[b]).wait()
                pltpu.sync_copy(rows_v.at[b],
                                out_hbm.at[pl.ds(base + t * _C, _C)])

                @pl.when(g < _NG - 1)
                def _refill():
                    fire(t + _NBUF, b)
            return _

        lax.fori_loop(0, _NG, group, None)

    return emb_kernel


_emb_kernel = _make_kernel()


@jax.jit
def kernel(x, table):
    idx = x.astype(jnp.int32).reshape(_NW, _NCHUNK, _C)
    out = _emb_kernel(table, idx)
    return out.reshape(x.shape[0], x.shape[1], EMB_DIM)
